# baseline (speedup 1.0000x reference)
import jax
import jax.numpy as jnp
import numpy as np
from jax.experimental import pallas as pl
from jax.experimental.pallas import tpu as pltpu

_BITS_I = np.array([2, 4, 6, 8, 12, 16])
_BITS_ROW = _BITS_I.astype(np.float32).reshape(1, 6)
_QN_ROW = (-(2.0 ** (_BITS_I - 1))).astype(np.float32).reshape(1, 6)
_QP_ROW = (2.0 ** (_BITS_I - 1) - 1.0).astype(np.float32).reshape(1, 6)

_T = 1024


def _body(v_ref, si_ref, snr_ref, gd_ref, gs_ref,
          W1_ref, b1_ref, W2_ref, b2_ref, Wds_ref, bds_ref, sd_ref, ss_ref,
          tbl_ref,
          vq_ref, sq_ref, bpl_ref, wd_ref, ws_ref, db_ref, sb_ref):
    v = v_ref[...]
    si = si_ref[...]
    snr = snr_ref[...]

    mag = jnp.sqrt(v[:, 0:1] ** 2 + v[:, 1:2] ** 2 + 1e-10)
    pin = jnp.concatenate(
        [v, snr, si[:, 2:3], mag, si[:, 0:1], si[:, 1:2],
         jnp.zeros_like(snr)], axis=-1)

    h = jnp.maximum(
        jnp.dot(pin, W1_ref[...], preferred_element_type=jnp.float32)
        + b1_ref[...], 0.0)
    h = jnp.maximum(
        jnp.dot(h, W2_ref[...], preferred_element_type=jnp.float32)
        + b2_ref[...], 0.0)
    logits = jnp.dot(h, Wds_ref[...], preferred_element_type=jnp.float32) \
        + bds_ref[...]

    lane6 = jax.lax.broadcasted_iota(jnp.int32, (v.shape[0], 6), 1)
    bits_row = tbl_ref[0:1, :]
    qn_row = tbl_ref[1:2, :]
    qp_row = tbl_ref[2:3, :]

    def gate_and_quant(z, s_row, x):
        idx = jnp.argmax(z, axis=-1, keepdims=True)
        w = jnp.where(lane6 == idx, 1.0, 0.0)
        s_sel = jnp.sum(w * s_row, axis=-1, keepdims=True)
        qn_sel = jnp.sum(w * qn_row, axis=-1, keepdims=True)
        qp_sel = jnp.sum(w * qp_row, axis=-1, keepdims=True)
        bits_sel = jnp.sum(w * bits_row, axis=-1, keepdims=True)
        xq = s_sel * jnp.round(jnp.clip(x / s_sel, qn_sel, qp_sel))
        return w, xq, bits_sel

    wd, vq, bits_d = gate_and_quant(logits[:, 0:6] + gd_ref[...], sd_ref[...], v)
    ws, sq, bits_s = gate_and_quant(logits[:, 6:12] + gs_ref[...], ss_ref[...], si)

    db = 2.0 * bits_d
    sb = 3.0 * bits_s
    vq_ref[...] = vq
    sq_ref[...] = sq
    wd_ref[...] = wd
    ws_ref[...] = ws
    db_ref[...] = db
    sb_ref[...] = sb
    bpl_ref[...] = db + sb


def kernel(v, side_info, local_snr, g_demod, g_side,
           W1, b1, W2, b2, Wd, bd, Ws, bs, s_demod, s_side):
    B, L, K, _ = v.shape
    N = B * L * K
    T = _T
    grid = (N // T,)

    v2 = v.reshape(N, 2)
    si2 = side_info.reshape(N, 3)
    snr2 = local_snr.reshape(N, 1)
    gd2 = g_demod.reshape(N, 6)
    gs2 = g_side.reshape(N, 6)

    Wds = jnp.concatenate([Wd, Ws], axis=1)
    bds = jnp.concatenate([bd, bs], axis=0).reshape(1, 12)
    b1r = b1.reshape(1, -1)
    b2r = b2.reshape(1, -1)
    sdr = s_demod.reshape(1, 6)
    ssr = s_side.reshape(1, 6)

    def tok(c):
        return pl.BlockSpec((T, c), lambda i: (i, 0))

    def full(shape):
        return pl.BlockSpec(shape, lambda i: (0, 0))

    out_shapes = (
        jax.ShapeDtypeStruct((N, 2), jnp.float32),
        jax.ShapeDtypeStruct((N, 3), jnp.float32),
        jax.ShapeDtypeStruct((N, 1), jnp.float32),
        jax.ShapeDtypeStruct((N, 6), jnp.float32),
        jax.ShapeDtypeStruct((N, 6), jnp.float32),
        jax.ShapeDtypeStruct((N, 1), jnp.float32),
        jax.ShapeDtypeStruct((N, 1), jnp.float32),
    )
    outs = pl.pallas_call(
        _body,
        grid=grid,
        in_specs=[tok(2), tok(3), tok(1), tok(6), tok(6),
                  full((8, 48)), full((1, 48)), full((48, 48)), full((1, 48)),
                  full((48, 12)), full((1, 12)), full((1, 6)), full((1, 6)),
                  full((3, 6))],
        out_specs=[tok(2), tok(3), tok(1), tok(6), tok(6), tok(1), tok(1)],
        out_shape=out_shapes,
        compiler_params=pltpu.CompilerParams(
            dimension_semantics=("parallel",),
            vmem_limit_bytes=100 * 1024 * 1024,
        ),
    )
    tbl = jnp.asarray(np.concatenate([_BITS_ROW, _QN_ROW, _QP_ROW], axis=0))
    outs = outs(v2, si2, snr2, gd2, gs2,
                jnp.pad(W1, ((0, 1), (0, 0))), b1r, W2, b2r, Wds, bds,
                sdr, ssr, tbl)

    vq, sq, bpl, wd, ws, db, sb = outs
    return (vq.reshape(B, L, K, 2),
            sq.reshape(B, L, K, 3),
            bpl.reshape(B, L, K),
            wd.reshape(B, L, K, 6),
            ws.reshape(B, L, K, 6),
            db.reshape(B, L, K),
            sb.reshape(B, L, K),)
